# Initial kernel scaffold; baseline (speedup 1.0000x reference)
import jax
import jax.numpy as jnp
from jax.experimental import pallas as pl


def kernel(label, freq):
    raise NotImplementedError("write your pallas kernel here")


# speedup vs baseline: 3.5118x; 3.5118x over previous
import functools

import jax
import jax.numpy as jnp
from jax import lax
from jax.experimental import pallas as pl
from jax.experimental.pallas import tpu as pltpu
from jax.experimental.pallas import tpu_sc as plsc

CLASSES = 19
PADC = 32
DECAY = 0.99
TEMPERATURE = 0.5
EPS = 1e-07
NEG_BIG = -1e30

NC = 2
NS = 16
NW = NC * NS
L = 16

N = 16 * 512 * 512
PER_W = N // NW
CHUNK = 16384
NCHUNK = PER_W // CHUNK

_mesh = plsc.VectorSubcoreMesh(core_axis_name="c", subcore_axis_name="s")
_params = pltpu.CompilerParams(needs_layout_passes=False)


def _worker_id():
    return lax.axis_index("s") * NC + lax.axis_index("c")


@functools.partial(
    pl.kernel,
    mesh=_mesh,
    compiler_params=_params,
    out_type=jax.ShapeDtypeStruct((NW, PADC), jnp.int32),
    scratch_types=[
        pltpu.VMEM((CHUNK,), jnp.int32),
        pltpu.VMEM((L * PADC,), jnp.int32),
        pltpu.VMEM((PADC,), jnp.int32),
    ],
)
def _hist_kernel(label_hbm, part_hbm, lbl_v, tab_v, cnt_v):
    wid = _worker_id()
    base = wid * PER_W
    zero16 = jnp.zeros((L,), jnp.int32)
    ones16 = jnp.ones((L,), jnp.int32)
    lane_off = lax.iota(jnp.int32, L) * PADC

    for r in range(L * PADC // L):
        tab_v[pl.ds(r * L, L)] = zero16

    for ch in range(NCHUNK):
        pltpu.sync_copy(label_hbm.at[pl.ds(base + ch * CHUNK, CHUNK)], lbl_v)

        def body(i, carry):
            v = lbl_v[pl.ds(i * L, L)]
            plsc.addupdate_scatter(tab_v, [lane_off + v], ones16)
            return carry

        lax.fori_loop(0, CHUNK // L, body, 0, unroll=8)

    c0 = jnp.zeros((L,), jnp.int32)
    c1 = jnp.zeros((L,), jnp.int32)
    for r in range(L):
        c0 = c0 + tab_v[pl.ds(r * PADC, L)]
        c1 = c1 + tab_v[pl.ds(r * PADC + L, L)]
    cnt_v[pl.ds(0, L)] = c0
    cnt_v[pl.ds(L, L)] = c1
    pltpu.sync_copy(cnt_v, part_hbm.at[wid])


@functools.partial(
    pl.kernel,
    mesh=_mesh,
    compiler_params=_params,
    out_type=(
        jax.ShapeDtypeStruct((N,), jnp.float32),
        jax.ShapeDtypeStruct((PADC,), jnp.float32),
    ),
    scratch_types=[
        pltpu.VMEM((NW, PADC), jnp.int32),
        pltpu.VMEM((PADC,), jnp.float32),
        pltpu.VMEM((PADC,), jnp.float32),
        pltpu.VMEM((CHUNK,), jnp.int32),
        pltpu.VMEM((CHUNK,), jnp.float32),
    ],
)
def _gather_kernel(label_hbm, part_hbm, freq_hbm, weight_hbm, fnew_hbm,
                   part_v, freq_v, prob_v, lbl_v, w_v):
    wid = _worker_id()
    base = wid * PER_W

    pltpu.sync_copy(part_hbm, part_v)
    pltpu.sync_copy(freq_hbm, freq_v)

    c0 = jnp.zeros((L,), jnp.int32)
    c1 = jnp.zeros((L,), jnp.int32)
    for r in range(NW):
        c0 = c0 + part_v[r, pl.ds(0, L)]
        c1 = c1 + part_v[r, pl.ds(L, L)]

    inv_total = 1.0 / (float(N) + EPS)
    cf0 = c0.astype(jnp.float32) * inv_total
    cf1 = c1.astype(jnp.float32) * inv_total
    fn0 = (1.0 - DECAY) * cf0 + DECAY * freq_v[pl.ds(0, L)]
    fn1 = (1.0 - DECAY) * cf1 + DECAY * freq_v[pl.ds(L, L)]

    valid1 = lax.iota(jnp.int32, L) < (CLASSES - L)
    x0 = (1.0 - fn0) / TEMPERATURE
    x1 = (1.0 - fn1) / TEMPERATURE
    m = jnp.maximum(jnp.max(x0), jnp.max(jnp.where(valid1, x1, NEG_BIG)))
    e0 = jnp.exp(x0 - m)
    e1 = jnp.where(valid1, jnp.exp(x1 - m), 0.0)
    s = jnp.sum(e0) + jnp.sum(e1)
    p0 = e0 / s
    p1 = e1 / s
    pmax = jnp.maximum(jnp.max(p0), jnp.max(p1))
    pn0 = p0 / (pmax + EPS)
    pn1 = p1 / (pmax + EPS)
    prob_v[pl.ds(0, L)] = pn0
    prob_v[pl.ds(L, L)] = pn1

    @pl.when(wid == 0)
    def _():
        freq_v[pl.ds(0, L)] = fn0
        freq_v[pl.ds(L, L)] = fn1
        pltpu.sync_copy(freq_v, fnew_hbm)

    for ch in range(NCHUNK):
        pltpu.sync_copy(label_hbm.at[pl.ds(base + ch * CHUNK, CHUNK)], lbl_v)

        def body(i, carry):
            v = lbl_v[pl.ds(i * L, L)]
            w_v[pl.ds(i * L, L)] = plsc.load_gather(prob_v, [v])
            return carry

        lax.fori_loop(0, CHUNK // L, body, 0, unroll=8)
        pltpu.sync_copy(w_v, weight_hbm.at[pl.ds(base + ch * CHUNK, CHUNK)])


def kernel(label, freq):
    flat = jnp.reshape(label, (N,)).astype(jnp.int32)
    freq_pad = jnp.zeros((PADC,), jnp.float32).at[:CLASSES].set(
        freq.astype(jnp.float32))
    partials = _hist_kernel(flat)
    weight, fnew_pad = _gather_kernel(flat, partials, freq_pad)
    return weight, fnew_pad[:CLASSES]


# speedup vs baseline: 8.3409x; 2.3751x over previous
import functools

import jax
import jax.numpy as jnp
from jax import lax
from jax.experimental import pallas as pl
from jax.experimental.pallas import tpu as pltpu
from jax.experimental.pallas import tpu_sc as plsc

CLASSES = 19
PADC = 32
DECAY = 0.99
TEMPERATURE = 0.5
EPS = 1e-07
NEG_BIG = -1e30

NC = 2
NS = 16
NW = NC * NS
L = 16

N = 16 * 512 * 512
PER_W = N // NW
CHUNK = 16384
NCHUNK = PER_W // CHUNK

_mesh = plsc.VectorSubcoreMesh(core_axis_name="c", subcore_axis_name="s")
_params = pltpu.CompilerParams(needs_layout_passes=False)


def _worker_id():
    return lax.axis_index("s") * NC + lax.axis_index("c")


@functools.partial(
    pl.kernel,
    mesh=_mesh,
    compiler_params=_params,
    out_type=jax.ShapeDtypeStruct((NW, PADC), jnp.int32),
    scratch_types=[
        pltpu.VMEM((2, CHUNK), jnp.int32),
        pltpu.VMEM((L * PADC,), jnp.int32),
        pltpu.VMEM((PADC,), jnp.int32),
        pltpu.SemaphoreType.DMA,
        pltpu.SemaphoreType.DMA,
    ],
)
def _hist_kernel(label_hbm, part_hbm, lbl_v, tab_v, cnt_v, sem0, sem1):
    wid = _worker_id()
    base = wid * PER_W
    sems = (sem0, sem1)

    cps = [None, None]
    cps[0] = pltpu.async_copy(
        label_hbm.at[pl.ds(base, CHUNK)], lbl_v.at[0], sems[0])

    zero16 = jnp.zeros((L,), jnp.int32)
    ones16 = jnp.ones((L,), jnp.int32)
    lane_off = lax.iota(jnp.int32, L) * PADC

    for r in range(PADC):
        tab_v[pl.ds(r * L, L)] = zero16

    for ch in range(NCHUNK):
        b = ch & 1
        if ch + 1 < NCHUNK:
            cps[1 - b] = pltpu.async_copy(
                label_hbm.at[pl.ds(base + (ch + 1) * CHUNK, CHUNK)],
                lbl_v.at[1 - b], sems[1 - b])
        cps[b].wait()

        @plsc.parallel_loop(0, CHUNK, step=L, unroll=8)
        def _(i):
            v = lbl_v[b, pl.ds(i, L)]
            plsc.addupdate_scatter(tab_v, [lane_off + v], ones16)

    c0 = jnp.zeros((L,), jnp.int32)
    c1 = jnp.zeros((L,), jnp.int32)
    for r in range(L):
        c0 = c0 + tab_v[pl.ds(r * PADC, L)]
        c1 = c1 + tab_v[pl.ds(r * PADC + L, L)]
    cnt_v[pl.ds(0, L)] = c0
    cnt_v[pl.ds(L, L)] = c1
    pltpu.sync_copy(cnt_v, part_hbm.at[wid])


@functools.partial(
    pl.kernel,
    mesh=_mesh,
    compiler_params=_params,
    out_type=(
        jax.ShapeDtypeStruct((N,), jnp.float32),
        jax.ShapeDtypeStruct((PADC,), jnp.float32),
    ),
    scratch_types=[
        pltpu.VMEM((NW, PADC), jnp.int32),
        pltpu.VMEM((PADC,), jnp.float32),
        pltpu.VMEM((PADC,), jnp.float32),
        pltpu.VMEM((2, CHUNK), jnp.int32),
        pltpu.VMEM((2, CHUNK), jnp.float32),
        pltpu.SemaphoreType.DMA,
        pltpu.SemaphoreType.DMA,
        pltpu.SemaphoreType.DMA,
        pltpu.SemaphoreType.DMA,
    ],
)
def _gather_kernel(label_hbm, part_hbm, freq_hbm, weight_hbm, fnew_hbm,
                   part_v, freq_v, prob_v, lbl_v, w_v,
                   si0, si1, so0, so1):
    wid = _worker_id()
    base = wid * PER_W
    sin = (si0, si1)
    sout = (so0, so1)

    in_cp = [None, None]
    in_cp[0] = pltpu.async_copy(
        label_hbm.at[pl.ds(base, CHUNK)], lbl_v.at[0], sin[0])

    pltpu.sync_copy(part_hbm, part_v)
    pltpu.sync_copy(freq_hbm, freq_v)

    c0 = jnp.zeros((L,), jnp.int32)
    c1 = jnp.zeros((L,), jnp.int32)
    for r in range(NW):
        c0 = c0 + part_v[r, pl.ds(0, L)]
        c1 = c1 + part_v[r, pl.ds(L, L)]

    inv_total = 1.0 / (float(N) + EPS)
    cf0 = c0.astype(jnp.float32) * inv_total
    cf1 = c1.astype(jnp.float32) * inv_total
    fn0 = (1.0 - DECAY) * cf0 + DECAY * freq_v[pl.ds(0, L)]
    fn1 = (1.0 - DECAY) * cf1 + DECAY * freq_v[pl.ds(L, L)]

    valid1 = lax.iota(jnp.int32, L) < (CLASSES - L)
    x0 = (1.0 - fn0) / TEMPERATURE
    x1 = (1.0 - fn1) / TEMPERATURE
    m = jnp.maximum(jnp.max(x0), jnp.max(jnp.where(valid1, x1, NEG_BIG)))
    e0 = jnp.exp(x0 - m)
    e1 = jnp.where(valid1, jnp.exp(x1 - m), 0.0)
    s = jnp.sum(e0) + jnp.sum(e1)
    p0 = e0 / s
    p1 = e1 / s
    pmax = jnp.maximum(jnp.max(p0), jnp.max(p1))
    pn0 = p0 / (pmax + EPS)
    pn1 = p1 / (pmax + EPS)
    prob_v[pl.ds(0, L)] = pn0
    prob_v[pl.ds(L, L)] = pn1

    @pl.when(wid == 0)
    def _():
        freq_v[pl.ds(0, L)] = fn0
        freq_v[pl.ds(L, L)] = fn1
        pltpu.sync_copy(freq_v, fnew_hbm)

    out_cp = [None, None]
    for ch in range(NCHUNK):
        b = ch & 1
        if ch + 1 < NCHUNK:
            in_cp[1 - b] = pltpu.async_copy(
                label_hbm.at[pl.ds(base + (ch + 1) * CHUNK, CHUNK)],
                lbl_v.at[1 - b], sin[1 - b])
        in_cp[b].wait()
        if out_cp[b] is not None:
            out_cp[b].wait()

        @plsc.parallel_loop(0, CHUNK, step=L, unroll=8)
        def _(i):
            v = lbl_v[b, pl.ds(i, L)]
            w_v[b, pl.ds(i, L)] = plsc.load_gather(prob_v, [v])

        out_cp[b] = pltpu.async_copy(
            w_v.at[b], weight_hbm.at[pl.ds(base + ch * CHUNK, CHUNK)],
            sout[b])

    for b in range(2):
        if out_cp[b] is not None:
            out_cp[b].wait()


def kernel(label, freq):
    flat = jnp.reshape(label, (N,)).astype(jnp.int32)
    freq_pad = jnp.zeros((PADC,), jnp.float32).at[:CLASSES].set(
        freq.astype(jnp.float32))
    partials = _hist_kernel(flat)
    weight, fnew_pad = _gather_kernel(flat, partials, freq_pad)
    return weight, fnew_pad[:CLASSES]
